# initial kernel scaffold (unmeasured)
import jax
import jax.numpy as jnp
from jax import lax
from jax.experimental import pallas as pl
from jax.experimental.pallas import tpu as pltpu

N_DEV = 32
BLK = 32


def kernel(x, w_mat):
    m_total, k_local = x.shape
    k_total, n = w_mat.shape

    def body(x_ref, w_ref, out_ref, recv_ref, send_sems, recv_sems):
        me = lax.axis_index("i")

        rdmas = []
        for d in range(1, N_DEV):
            t = (me + d) % N_DEV
            rdma = pltpu.make_async_remote_copy(
                src_ref=x_ref.at[pl.ds(t * BLK, BLK), :],
                dst_ref=recv_ref.at[d],
                send_sem=send_sems.at[d],
                recv_sem=recv_sems.at[d],
                device_id=(t,),
                device_id_type=pl.DeviceIdType.MESH,
            )
            rdma.start()
            rdmas.append(rdma)

        acc = jnp.dot(
            x_ref[pl.ds(me * BLK, BLK), :],
            w_ref[pl.ds(me * BLK, BLK), :],
            preferred_element_type=jnp.float32,
        )
        for d in range(1, N_DEV):
            s = (me - d) % N_DEV
            rdmas[d - 1].wait_recv()
            acc = acc + jnp.dot(
                recv_ref[d],
                w_ref[pl.ds(s * BLK, BLK), :],
                preferred_element_type=jnp.float32,
            )
        for r in rdmas:
            r.wait_send()
        out_ref[:, :] = acc

    return pl.pallas_call(
        body,
        out_shape=jax.ShapeDtypeStruct((BLK, n), jnp.float32),
        in_specs=[
            pl.BlockSpec(memory_space=pltpu.VMEM),
            pl.BlockSpec(memory_space=pltpu.VMEM),
        ],
        out_specs=pl.BlockSpec(memory_space=pltpu.VMEM),
        scratch_shapes=[
            pltpu.VMEM((N_DEV, BLK, BLK), x.dtype),
            pltpu.SemaphoreType.DMA((N_DEV,)),
            pltpu.SemaphoreType.DMA((N_DEV,)),
        ],
        compiler_params=pltpu.CompilerParams(collective_id=0),
    )(x, w_mat)


# baseline (device time: 24655 ns/iter reference)
import jax
import jax.numpy as jnp
from jax import lax
from jax.experimental import pallas as pl
from jax.experimental.pallas import tpu as pltpu

N_DEV = 32
BLK = 32


def kernel(x, w_mat):
    m_total, k_local = x.shape
    k_total, n = w_mat.shape

    def body(x_ref, w_ref, out_ref, recv_ref, send_sems, recv_sems):
        me = lax.axis_index("i")

        rdmas = []
        for d in range(1, N_DEV):
            t = (me + d) % N_DEV
            rdma = pltpu.make_async_remote_copy(
                src_ref=x_ref.at[pl.ds(t * BLK, BLK), :],
                dst_ref=recv_ref.at[d],
                send_sem=send_sems.at[d],
                recv_sem=recv_sems.at[d],
                device_id=(t,),
                device_id_type=pl.DeviceIdType.MESH,
            )
            rdma.start()
            rdmas.append(rdma)

        acc = jnp.dot(
            x_ref[pl.ds(me * BLK, BLK), :],
            w_ref[pl.ds(me * BLK, BLK), :],
            preferred_element_type=jnp.float32,
        )
        for d in range(1, N_DEV):
            s = (me - d) % N_DEV
            rdmas[d - 1].wait_recv()
            acc = acc + jnp.dot(
                recv_ref[d],
                w_ref[pl.ds(s * BLK, BLK), :],
                preferred_element_type=jnp.float32,
            )
        for r in rdmas:
            r.wait_send()
        out_ref[:, :] = acc

    return pl.pallas_call(
        body,
        out_shape=jax.ShapeDtypeStruct((BLK, n), jnp.float32),
        in_specs=[
            pl.BlockSpec(memory_space=pltpu.VMEM),
            pl.BlockSpec(memory_space=pltpu.VMEM),
        ],
        out_specs=pl.BlockSpec(memory_space=pltpu.VMEM),
        scratch_shapes=[
            pltpu.VMEM((N_DEV, BLK, BLK), x.dtype),
            pltpu.SemaphoreType.DMA((N_DEV,)),
            pltpu.SemaphoreType.DMA((N_DEV,)),
        ],
    )(x, w_mat)
